# baseline (speedup 1.0000x reference)
import jax
import jax.numpy as jnp
import numpy as np
from jax.experimental import pallas as pl
from jax.experimental.pallas import tpu as pltpu

_K_SUBSET = 16
_EPS = float(np.finfo(np.float32).tiny)


def _score_block(q_ref, nt_ref, g_ref, out_ref):
    qk = q_ref[...]
    ntk = nt_ref[...]
    gk = g_ref[...]
    accs = [None] * 16
    for t in range(2):
        for v in range(16):
            for s in range(8):
                d0 = t * 128 + v * 8 + s
                dif = qk[:, d0:d0 + 1] - ntk[d0:d0 + 1, :]
                sq = dif * dif
                idx = t * 8 + s
                accs[idx] = sq if v == 0 else accs[idx] + sq
    parts = []
    for t in range(2):
        a = [accs[t * 8 + s] for s in range(8)]
        b = [a[i] + a[i + 4] for i in range(4)]
        c = [b[i] + b[i + 2] for i in range(2)]
        parts.append(c[0] + c[1])
    l2 = parts[0] + parts[1]
    out_ref[...] = -l2 + gk


def _row_sum(e):
    bq = e.shape[0]
    e3 = e.reshape(bq, 256, 8)
    acc = e3[:, 0, :]
    for m in range(1, 256):
        acc = acc + e3[:, m, :]
    a = acc[:, :4] + acc[:, 4:]
    a = a[:, :2] + a[:, 2:]
    return a[:, 0:1] + a[:, 1:2]


def _subset_block(s_ref, out_ref):
    scores = s_ref[...]
    khot = jnp.zeros_like(scores)
    onehot = jnp.zeros_like(scores)
    for _ in range(_K_SUBSET):
        mask = jnp.maximum(1.0 - onehot, _EPS)
        scores = scores + jnp.log(mask)
        m = jnp.max(scores, axis=1, keepdims=True)
        e = jnp.exp(scores - m)
        onehot = e / _row_sum(e)
        khot = khot + onehot
    out_ref[...] = khot


@jax.jit
def kernel(query, neighbors, gumbel):
    Q, d = query.shape
    K = neighbors.shape[0]
    nt = neighbors.T
    BQ, BK = 64, 512
    scores = pl.pallas_call(
        _score_block,
        grid=(Q // BQ, K // BK),
        in_specs=[
            pl.BlockSpec((BQ, d), lambda i, j: (i, 0)),
            pl.BlockSpec((d, BK), lambda i, j: (0, j)),
            pl.BlockSpec((BQ, BK), lambda i, j: (i, j)),
        ],
        out_specs=pl.BlockSpec((BQ, BK), lambda i, j: (i, j)),
        out_shape=jax.ShapeDtypeStruct((Q, K), jnp.float32),
        compiler_params=pltpu.CompilerParams(
            dimension_semantics=("parallel", "parallel"),
        ),
    )(query, nt, gumbel)

    n_blocks = 2
    bq = Q // n_blocks
    return pl.pallas_call(
        _subset_block,
        grid=(n_blocks,),
        in_specs=[pl.BlockSpec((bq, K), lambda i: (i, 0))],
        out_specs=pl.BlockSpec((bq, K), lambda i: (i, 0)),
        out_shape=jax.ShapeDtypeStruct((Q, K), jnp.float32),
        compiler_params=pltpu.CompilerParams(
            dimension_semantics=("parallel",),
        ),
    )(scores)


# speedup vs baseline: 5.3475x; 5.3475x over previous
import jax
import jax.numpy as jnp
import numpy as np
from jax.experimental import pallas as pl
from jax.experimental.pallas import tpu as pltpu

_K_SUBSET = 16
_EPS = float(np.finfo(np.float32).tiny)


def _score_block(q_ref, nt_ref, g_ref, out_ref):
    qk = q_ref[...]
    ntk = nt_ref[...]
    gk = g_ref[...]
    accs = [None] * 16
    for t in range(2):
        for v in range(16):
            for s in range(8):
                d0 = t * 128 + v * 8 + s
                dif = qk[:, d0:d0 + 1] - ntk[d0:d0 + 1, :]
                sq = dif * dif
                idx = t * 8 + s
                accs[idx] = sq if v == 0 else accs[idx] + sq
    parts = []
    for t in range(2):
        a = [accs[t * 8 + s] for s in range(8)]
        b = [a[i] + a[i + 4] for i in range(4)]
        c = [b[i] + b[i + 2] for i in range(2)]
        parts.append(c[0] + c[1])
    l2 = parts[0] + parts[1]
    out_ref[...] = -l2 + gk


def _col_sum_t(e):
    acc = e[0:8, :]
    for m in range(1, 256):
        acc = acc + e[8 * m:8 * m + 8, :]
    a = acc[0:4, :] + acc[4:8, :]
    a = a[0:2, :] + a[2:4, :]
    return a[0:1, :] + a[1:2, :]


def _subset_block(s_ref, out_ref):
    scores = s_ref[...]
    khot = jnp.zeros_like(scores)
    onehot = jnp.zeros_like(scores)
    for _ in range(_K_SUBSET):
        mask = jnp.maximum(1.0 - onehot, _EPS)
        scores = scores + jnp.log(mask)
        m = jnp.max(scores, axis=0, keepdims=True)
        e = jnp.exp(scores - m)
        onehot = e / _col_sum_t(e)
        khot = khot + onehot
    out_ref[...] = khot


@jax.jit
def kernel(query, neighbors, gumbel):
    Q, d = query.shape
    K = neighbors.shape[0]
    nt = neighbors.T
    BQ, BK = 64, 512
    scores = pl.pallas_call(
        _score_block,
        grid=(Q // BQ, K // BK),
        in_specs=[
            pl.BlockSpec((BQ, d), lambda i, j: (i, 0)),
            pl.BlockSpec((d, BK), lambda i, j: (0, j)),
            pl.BlockSpec((BQ, BK), lambda i, j: (i, j)),
        ],
        out_specs=pl.BlockSpec((BQ, BK), lambda i, j: (i, j)),
        out_shape=jax.ShapeDtypeStruct((Q, K), jnp.float32),
        compiler_params=pltpu.CompilerParams(
            dimension_semantics=("parallel", "parallel"),
        ),
    )(query, nt, gumbel)

    n_blocks = 2
    bq = Q // n_blocks
    khot_t = pl.pallas_call(
        _subset_block,
        grid=(n_blocks,),
        in_specs=[pl.BlockSpec((K, bq), lambda i: (0, i))],
        out_specs=pl.BlockSpec((K, bq), lambda i: (0, i)),
        out_shape=jax.ShapeDtypeStruct((K, Q), jnp.float32),
        compiler_params=pltpu.CompilerParams(
            dimension_semantics=("parallel",),
        ),
    )(scores.T)
    return khot_t.T
